# baseline (speedup 1.0000x reference)
import functools

import jax
import jax.numpy as jnp
from jax import lax
from jax.experimental import pallas as pl
from jax.experimental.pallas import tpu as pltpu
from jax.experimental.pallas import tpu_sc as plsc

_N = 10000
_E = 320000
_DF = 128
_DI = 64

_NC = 2
_NS = 16
_NW = _NC * _NS
_CHUNK = 128
_EPW = 10240
_E_PAD = _NW * _EPW
_PAD_ROWS = 16
_N_ACC = _N + 112
_RPT = _N_ACC // _NS


def _leaky(v):
    return jnp.where(v >= 0, v, 0.01 * v)


def _make_sc_scatter(d, chunk):
    nchunk = _EPW // chunk
    hch = 40
    mesh = plsc.VectorSubcoreMesh(core_axis_name="c", subcore_axis_name="s",
                                  num_cores=_NC, num_subcores=_NS)

    params = (pltpu.CompilerParams(use_tc_tiling_on_sc=False)
              if d % 128 != 0 else None)

    @functools.partial(
        pl.kernel,
        out_type=jax.ShapeDtypeStruct((_NC, _N_ACC, d), jnp.float32),
        mesh=mesh,
        compiler_params=params,
        scratch_types=[
            pltpu.VMEM((hch, chunk), jnp.int32),
            pltpu.VMEM((hch, chunk), jnp.int32),
            [pltpu.VMEM((chunk, d), jnp.float32) for _ in range(4)],
            pltpu.VMEM_SHARED((_N_ACC, d), jnp.float32),
            [pltpu.SemaphoreType.DMA for _ in range(4)],
            [pltpu.SemaphoreType.DMA for _ in range(4)],
        ],
    )
    def sc_scatter(xw_hbm, src_hbm, dst_hbm, zeros_hbm, out_hbm,
                   src_v, dst_v, rows, acc_sh, gsem, ssem):
        c = lax.axis_index("c")
        s = lax.axis_index("s")
        w = c * _NS + s

        def gather(j, b):
            pltpu.async_copy(xw_hbm.at[src_v.at[j]], rows[b], gsem[b])

        def gather_wait(b):
            pltpu.make_async_copy(xw_hbm.at[src_v.at[0]], rows[b],
                                  gsem[b]).wait()

        def scatter(j, b):
            pltpu.async_copy(rows[b], acc_sh.at[dst_v.at[j]], ssem[b],
                             add=True)

        def scatter_wait(b):
            pltpu.make_async_copy(rows[b], acc_sh.at[dst_v.at[0]],
                                  ssem[b]).wait()

        pltpu.sync_copy(zeros_hbm.at[pl.ds(s * _RPT, _RPT)],
                        acc_sh.at[pl.ds(s * _RPT, _RPT)])
        plsc.subcore_barrier()

        for p in range(nchunk // hch):
            base = w * nchunk + p * hch
            pltpu.sync_copy(src_hbm.at[pl.ds(base, hch)], src_v)
            pltpu.sync_copy(dst_hbm.at[pl.ds(base, hch)], dst_v)
            gather(0, 0)
            gather(1, 1)
            gather_wait(0)
            scatter(0, 0)
            gather(2, 2)
            gather_wait(1)
            scatter(1, 1)
            gather(3, 3)

            def body(g, carry):
                j = 4 * g + 2
                for i in range(4):
                    b = (2 + i) % 4
                    gather_wait(b)
                    scatter(j + i, b)
                    scatter_wait((b + 2) % 4)
                    gather(j + i + 2, (b + 2) % 4)
                return carry

            lax.fori_loop(0, (hch - 4) // 4, body, 0)
            gather_wait(2)
            scatter(hch - 2, 2)
            gather_wait(3)
            scatter(hch - 1, 3)
            for b in range(4):
                scatter_wait(b)
        plsc.subcore_barrier()
        pltpu.sync_copy(acc_sh.at[pl.ds(s * _RPT, _RPT)],
                        out_hbm.at[c, pl.ds(s * _RPT, _RPT)])

    return sc_scatter


_sc_scatter_128 = _make_sc_scatter(_DF, _CHUNK // 2)
_sc_scatter_64 = _make_sc_scatter(_DI, _CHUNK)


_BLK = 1000


def _dot(a, b):
    return jnp.dot(a, b, preferred_element_type=jnp.float32)


def _pre_body(x_ref, id_ref, c1_ref, l1w_ref, l1b_ref, xw_ref, xhat_ref):
    x = x_ref[...]
    nrm = jnp.sqrt(jnp.sum(x * x, axis=1, keepdims=True))
    xn = x / jnp.maximum(nrm, 1e-12)
    xw_ref[...] = _dot(xn, c1_ref[...])
    xhat_ref[...] = _leaky(_dot(xn, l1w_ref[...]) + l1b_ref[...]) + id_ref[...]


def _mid_body(h0_ref, h1_ref, xhat_ref, id_ref, g1w_ref, g1b_ref,
              c2_ref, l2w_ref, l2b_ref, xw2_ref, xhat2_ref):
    h = _leaky(h0_ref[0] + h1_ref[0])
    x2 = _leaky(_dot(h, g1w_ref[...]) + g1b_ref[...] + xhat_ref[...])
    xw2_ref[...] = _dot(x2, c2_ref[...])
    xhat2_ref[...] = _leaky(_dot(x2, l2w_ref[...]) + l2b_ref[...]) + id_ref[...]


def _post_body(h0_ref, h1_ref, xhat2_ref, g2w_ref, g2b_ref, o_ref):
    h = _leaky(h0_ref[0] + h1_ref[0])
    o_ref[...] = _leaky(_dot(h, g2w_ref[...]) + g2b_ref[...] + xhat2_ref[...])


def _row_spec(d):
    return pl.BlockSpec((_BLK, d), lambda i: (i, 0))


def _part_spec(core, d):
    return pl.BlockSpec((1, _BLK, d), lambda i, _c=core: (_c, i, 0))


def _full_spec(r, c):
    return pl.BlockSpec((r, c), lambda i: (0, 0))


_GRID = _N // _BLK

_pre_call = pl.pallas_call(
    _pre_body,
    grid=(_GRID,),
    in_specs=[_row_spec(_DF), _row_spec(_DI), _full_spec(_DF, _DF),
              _full_spec(_DF, _DI), _full_spec(1, _DI)],
    out_specs=[_row_spec(_DF), _row_spec(_DI)],
    out_shape=[jax.ShapeDtypeStruct((_N, _DF), jnp.float32),
               jax.ShapeDtypeStruct((_N, _DI), jnp.float32)],
)

_mid_call = pl.pallas_call(
    _mid_body,
    grid=(_GRID,),
    in_specs=[_part_spec(0, _DF), _part_spec(1, _DF), _row_spec(_DI),
              _row_spec(_DI), _full_spec(_DF, _DI), _full_spec(1, _DI),
              _full_spec(_DI, _DI), _full_spec(_DI, _DI), _full_spec(1, _DI)],
    out_specs=[_row_spec(_DI), _row_spec(_DI)],
    out_shape=[jax.ShapeDtypeStruct((_N, _DI), jnp.float32),
               jax.ShapeDtypeStruct((_N, _DI), jnp.float32)],
)

_post_call = pl.pallas_call(
    _post_body,
    grid=(_GRID,),
    in_specs=[_part_spec(0, _DI), _part_spec(1, _DI), _row_spec(_DI),
              _full_spec(_DI, _DI), _full_spec(1, _DI)],
    out_specs=_row_spec(_DI),
    out_shape=jax.ShapeDtypeStruct((_N, _DI), jnp.float32),
)


def kernel(features, id_embedding, edge_index, conv1_W, lin1_W, lin1_b,
           g1_W, g1_b, conv2_W, lin2_W, lin2_b, g2_W, g2_b):
    ei = edge_index.astype(jnp.int32)
    pad = _E_PAD - _E
    ar = jnp.arange(pad, dtype=jnp.int32)
    src = jnp.concatenate([ei[0], ar % _N])
    dst = jnp.concatenate([ei[1], _N + (ar % _PAD_ROWS)])
    src1 = src.reshape(_E_PAD // (_CHUNK // 2), _CHUNK // 2)
    dst1 = dst.reshape(_E_PAD // (_CHUNK // 2), _CHUNK // 2)
    src2 = src.reshape(_E_PAD // _CHUNK, _CHUNK)
    dst2 = dst.reshape(_E_PAD // _CHUNK, _CHUNK)
    z128 = jnp.zeros((_N_ACC, _DF), jnp.float32)
    z64 = jnp.zeros((_N_ACC, _DI), jnp.float32)
    l1b = lin1_b.reshape(1, _DI)
    g1b = g1_b.reshape(1, _DI)
    l2b = lin2_b.reshape(1, _DI)
    g2b = g2_b.reshape(1, _DI)

    xw1, xhat1 = _pre_call(features, id_embedding, conv1_W, lin1_W, l1b)
    hp1 = _sc_scatter_128(xw1, src1, dst1, z128)
    xw2, xhat2 = _mid_call(hp1, hp1, xhat1, id_embedding,
                           g1_W, g1b, conv2_W, lin2_W, l2b)
    hp2 = _sc_scatter_64(xw2, src2, dst2, z64)
    out = _post_call(hp2, hp2, xhat2, g2_W, g2b)
    return out


# speedup vs baseline: 1.0752x; 1.0752x over previous
import functools

import jax
import jax.numpy as jnp
from jax import lax
from jax.experimental import pallas as pl
from jax.experimental.pallas import tpu as pltpu
from jax.experimental.pallas import tpu_sc as plsc

_N = 10000
_E = 320000
_DF = 128
_DI = 64

_NC = 2
_NS = 16
_NW = _NC * _NS
_CHUNK = 128
_EPW = 10240
_E_PAD = _NW * _EPW
_PAD_ROWS = 16
_N_ACC = _N + 112
_RPT = _N_ACC // _NS


def _leaky(v):
    return jnp.where(v >= 0, v, 0.01 * v)


def _make_sc_scatter(d, chunk, nbuf):
    nchunk = _EPW // chunk
    hch = 40
    mesh = plsc.VectorSubcoreMesh(core_axis_name="c", subcore_axis_name="s",
                                  num_cores=_NC, num_subcores=_NS)

    params = (pltpu.CompilerParams(use_tc_tiling_on_sc=False)
              if d % 128 != 0 else None)

    @functools.partial(
        pl.kernel,
        out_type=jax.ShapeDtypeStruct((_NC, _N_ACC, d), jnp.float32),
        mesh=mesh,
        compiler_params=params,
        scratch_types=[
            pltpu.VMEM((hch, chunk), jnp.int32),
            pltpu.VMEM((hch, chunk), jnp.int32),
            [pltpu.VMEM((chunk, d), jnp.float32) for _ in range(nbuf)],
            pltpu.VMEM_SHARED((_N_ACC, d), jnp.float32),
            [pltpu.SemaphoreType.DMA for _ in range(nbuf)],
            [pltpu.SemaphoreType.DMA for _ in range(nbuf)],
        ],
    )
    def sc_scatter(xw_hbm, src_hbm, dst_hbm, zeros_hbm, out_hbm,
                   src_v, dst_v, rows, acc_sh, gsem, ssem):
        c = lax.axis_index("c")
        s = lax.axis_index("s")
        w = c * _NS + s

        def gather(j, b):
            pltpu.async_copy(xw_hbm.at[src_v.at[j]], rows[b], gsem[b])

        def gather_wait(b):
            pltpu.make_async_copy(xw_hbm.at[src_v.at[0]], rows[b],
                                  gsem[b]).wait()

        def scatter(j, b):
            pltpu.async_copy(rows[b], acc_sh.at[dst_v.at[j]], ssem[b],
                             add=True)

        def scatter_wait(b):
            pltpu.make_async_copy(rows[b], acc_sh.at[dst_v.at[0]],
                                  ssem[b]).wait()

        pltpu.sync_copy(zeros_hbm.at[pl.ds(s * _RPT, _RPT)],
                        acc_sh.at[pl.ds(s * _RPT, _RPT)])
        plsc.subcore_barrier()

        for p in range(nchunk // hch):
            base = w * nchunk + p * hch
            pltpu.sync_copy(src_hbm.at[pl.ds(base, hch)], src_v)
            pltpu.sync_copy(dst_hbm.at[pl.ds(base, hch)], dst_v)
            if nbuf == 2:
                gather(0, 0)
                gather(1, 1)

                def body(g, carry):
                    j = 2 * g
                    for i in range(2):
                        gather_wait(i)
                        scatter(j + i, i)
                        scatter_wait(i)
                        gather(j + i + 2, i)
                    return carry

                lax.fori_loop(0, hch // 2 - 1, body, 0)
                gather_wait(0)
                scatter(hch - 2, 0)
                gather_wait(1)
                scatter(hch - 1, 1)
                scatter_wait(0)
                scatter_wait(1)
            else:
                gather(0, 0)
                gather(1, 1)
                gather_wait(0)
                scatter(0, 0)
                gather(2, 2)
                gather_wait(1)
                scatter(1, 1)
                gather(3, 3)

                def body(g, carry):
                    j = 4 * g + 2
                    for i in range(4):
                        b = (2 + i) % 4
                        gather_wait(b)
                        scatter(j + i, b)
                        scatter_wait((b + 2) % 4)
                        gather(j + i + 2, (b + 2) % 4)
                    return carry

                lax.fori_loop(0, (hch - 4) // 4, body, 0)
                gather_wait(2)
                scatter(hch - 2, 2)
                gather_wait(3)
                scatter(hch - 1, 3)
                for b in range(4):
                    scatter_wait(b)
        plsc.subcore_barrier()
        pltpu.sync_copy(acc_sh.at[pl.ds(s * _RPT, _RPT)],
                        out_hbm.at[c, pl.ds(s * _RPT, _RPT)])

    return sc_scatter


_sc_scatter_128 = _make_sc_scatter(_DF, _CHUNK, 2)
_sc_scatter_64 = _make_sc_scatter(_DI, _CHUNK, 4)


_BLK = 1000


def _dot(a, b):
    return jnp.dot(a, b, preferred_element_type=jnp.float32)


def _pre_body(x_ref, id_ref, c1_ref, l1w_ref, l1b_ref, xw_ref, xhat_ref):
    x = x_ref[...]
    nrm = jnp.sqrt(jnp.sum(x * x, axis=1, keepdims=True))
    xn = x / jnp.maximum(nrm, 1e-12)
    xw_ref[...] = _dot(xn, c1_ref[...])
    xhat_ref[...] = _leaky(_dot(xn, l1w_ref[...]) + l1b_ref[...]) + id_ref[...]


def _mid_body(h0_ref, h1_ref, xhat_ref, id_ref, g1w_ref, g1b_ref,
              c2_ref, l2w_ref, l2b_ref, xw2_ref, xhat2_ref):
    h = _leaky(h0_ref[0] + h1_ref[0])
    x2 = _leaky(_dot(h, g1w_ref[...]) + g1b_ref[...] + xhat_ref[...])
    xw2_ref[...] = _dot(x2, c2_ref[...])
    xhat2_ref[...] = _leaky(_dot(x2, l2w_ref[...]) + l2b_ref[...]) + id_ref[...]


def _post_body(h0_ref, h1_ref, xhat2_ref, g2w_ref, g2b_ref, o_ref):
    h = _leaky(h0_ref[0] + h1_ref[0])
    o_ref[...] = _leaky(_dot(h, g2w_ref[...]) + g2b_ref[...] + xhat2_ref[...])


def _row_spec(d):
    return pl.BlockSpec((_BLK, d), lambda i: (i, 0))


def _part_spec(core, d):
    return pl.BlockSpec((1, _BLK, d), lambda i, _c=core: (_c, i, 0))


def _full_spec(r, c):
    return pl.BlockSpec((r, c), lambda i: (0, 0))


_GRID = _N // _BLK

_pre_call = pl.pallas_call(
    _pre_body,
    grid=(_GRID,),
    in_specs=[_row_spec(_DF), _row_spec(_DI), _full_spec(_DF, _DF),
              _full_spec(_DF, _DI), _full_spec(1, _DI)],
    out_specs=[_row_spec(_DF), _row_spec(_DI)],
    out_shape=[jax.ShapeDtypeStruct((_N, _DF), jnp.float32),
               jax.ShapeDtypeStruct((_N, _DI), jnp.float32)],
)

_mid_call = pl.pallas_call(
    _mid_body,
    grid=(_GRID,),
    in_specs=[_part_spec(0, _DF), _part_spec(1, _DF), _row_spec(_DI),
              _row_spec(_DI), _full_spec(_DF, _DI), _full_spec(1, _DI),
              _full_spec(_DI, _DI), _full_spec(_DI, _DI), _full_spec(1, _DI)],
    out_specs=[_row_spec(_DI), _row_spec(_DI)],
    out_shape=[jax.ShapeDtypeStruct((_N, _DI), jnp.float32),
               jax.ShapeDtypeStruct((_N, _DI), jnp.float32)],
)

_post_call = pl.pallas_call(
    _post_body,
    grid=(_GRID,),
    in_specs=[_part_spec(0, _DI), _part_spec(1, _DI), _row_spec(_DI),
              _full_spec(_DI, _DI), _full_spec(1, _DI)],
    out_specs=_row_spec(_DI),
    out_shape=jax.ShapeDtypeStruct((_N, _DI), jnp.float32),
)


def kernel(features, id_embedding, edge_index, conv1_W, lin1_W, lin1_b,
           g1_W, g1_b, conv2_W, lin2_W, lin2_b, g2_W, g2_b):
    ei = edge_index.astype(jnp.int32)
    pad = _E_PAD - _E
    ar = jnp.arange(pad, dtype=jnp.int32)
    src = jnp.concatenate([ei[0], ar % _N]).reshape(_E_PAD // _CHUNK, _CHUNK)
    dst = jnp.concatenate([ei[1], _N + (ar % _PAD_ROWS)]).reshape(
        _E_PAD // _CHUNK, _CHUNK)
    z128 = jnp.zeros((_N_ACC, _DF), jnp.float32)
    z64 = jnp.zeros((_N_ACC, _DI), jnp.float32)
    l1b = lin1_b.reshape(1, _DI)
    g1b = g1_b.reshape(1, _DI)
    l2b = lin2_b.reshape(1, _DI)
    g2b = g2_b.reshape(1, _DI)

    xw1, xhat1 = _pre_call(features, id_embedding, conv1_W, lin1_W, l1b)
    hp1 = _sc_scatter_128(xw1, src, dst, z128)
    xw2, xhat2 = _mid_call(hp1, hp1, xhat1, id_embedding,
                           g1_W, g1b, conv2_W, lin2_W, l2b)
    hp2 = _sc_scatter_64(xw2, src, dst, z64)
    out = _post_call(hp2, hp2, xhat2, g2_W, g2b)
    return out


# speedup vs baseline: 1.1029x; 1.0258x over previous
import functools

import jax
import jax.numpy as jnp
from jax import lax
from jax.experimental import pallas as pl
from jax.experimental.pallas import tpu as pltpu
from jax.experimental.pallas import tpu_sc as plsc

_N = 10000
_E = 320000
_DF = 128
_DI = 64

_NC = 2
_NS = 16
_NW = _NC * _NS
_CHUNK = 128
_EPW = 10240
_E_PAD = _NW * _EPW
_PAD_ROWS = 16
_N_ACC = _N + 112
_RPT = _N_ACC // _NS


def _leaky(v):
    return jnp.where(v >= 0, v, 0.01 * v)


def _make_sc_scatter(d, chunk, nbuf):
    nchunk = _EPW // chunk
    hch = 40
    mesh = plsc.VectorSubcoreMesh(core_axis_name="c", subcore_axis_name="s",
                                  num_cores=_NC, num_subcores=_NS)

    params = (pltpu.CompilerParams(use_tc_tiling_on_sc=False)
              if d % 128 != 0 else None)

    @functools.partial(
        pl.kernel,
        out_type=jax.ShapeDtypeStruct((_NC, _N_ACC, d), jnp.float32),
        mesh=mesh,
        compiler_params=params,
        scratch_types=[
            pltpu.VMEM((hch, chunk), jnp.int32),
            pltpu.VMEM((hch, chunk), jnp.int32),
            [pltpu.VMEM((chunk, d), jnp.float32) for _ in range(nbuf)],
            pltpu.VMEM_SHARED((_N_ACC, d), jnp.float32),
            [pltpu.SemaphoreType.DMA for _ in range(nbuf)],
            [pltpu.SemaphoreType.DMA for _ in range(nbuf)],
        ],
    )
    def sc_scatter(xw_hbm, src_hbm, dst_hbm, out_hbm,
                   src_v, dst_v, rows, acc_sh, gsem, ssem):
        c = lax.axis_index("c")
        s = lax.axis_index("s")
        w = c * _NS + s

        def gather(j, b):
            pltpu.async_copy(xw_hbm.at[src_v.at[j]], rows[b], gsem[b])

        def gather_wait(b):
            pltpu.make_async_copy(xw_hbm.at[src_v.at[0]], rows[b],
                                  gsem[b]).wait()

        def scatter(j, b):
            pltpu.async_copy(rows[b], acc_sh.at[dst_v.at[j]], ssem[b],
                             add=True)

        def scatter_wait(b):
            pltpu.make_async_copy(rows[b], acc_sh.at[dst_v.at[0]],
                                  ssem[b]).wait()

        def zbody(r, carry):
            for k in range(d // 16):
                rows[0][r, pl.ds(16 * k, 16)] = jnp.zeros((16,), jnp.float32)
            return carry

        lax.fori_loop(0, chunk, zbody, 0)
        for t in range(_RPT // chunk):
            pltpu.sync_copy(rows[0],
                            acc_sh.at[pl.ds(s * _RPT + t * chunk, chunk)])
        rem = _RPT % chunk
        if rem:
            pltpu.sync_copy(
                rows[0].at[pl.ds(0, rem)],
                acc_sh.at[pl.ds(s * _RPT + (_RPT // chunk) * chunk, rem)])
        plsc.subcore_barrier()

        for p in range(nchunk // hch):
            base = w * nchunk + p * hch
            pltpu.sync_copy(src_hbm.at[pl.ds(base, hch)], src_v)
            pltpu.sync_copy(dst_hbm.at[pl.ds(base, hch)], dst_v)
            if nbuf == 2:
                gather(0, 0)
                gather(1, 1)

                def body(g, carry):
                    j = 2 * g
                    for i in range(2):
                        gather_wait(i)
                        scatter(j + i, i)
                        scatter_wait(i)
                        gather(j + i + 2, i)
                    return carry

                lax.fori_loop(0, hch // 2 - 1, body, 0)
                gather_wait(0)
                scatter(hch - 2, 0)
                gather_wait(1)
                scatter(hch - 1, 1)
                scatter_wait(0)
                scatter_wait(1)
            else:
                gather(0, 0)
                gather(1, 1)
                gather_wait(0)
                scatter(0, 0)
                gather(2, 2)
                gather_wait(1)
                scatter(1, 1)
                gather(3, 3)

                def body(g, carry):
                    j = 4 * g + 2
                    for i in range(4):
                        b = (2 + i) % 4
                        gather_wait(b)
                        scatter(j + i, b)
                        scatter_wait((b + 2) % 4)
                        gather(j + i + 2, (b + 2) % 4)
                    return carry

                lax.fori_loop(0, (hch - 4) // 4, body, 0)
                gather_wait(2)
                scatter(hch - 2, 2)
                gather_wait(3)
                scatter(hch - 1, 3)
                for b in range(4):
                    scatter_wait(b)
        plsc.subcore_barrier()
        pltpu.sync_copy(acc_sh.at[pl.ds(s * _RPT, _RPT)],
                        out_hbm.at[c, pl.ds(s * _RPT, _RPT)])

    return sc_scatter


_sc_scatter_128 = _make_sc_scatter(_DF, _CHUNK, 2)
_sc_scatter_64 = _make_sc_scatter(_DI, _CHUNK, 4)


_BLK = 1000


def _dot(a, b):
    return jnp.dot(a, b, preferred_element_type=jnp.float32)


def _pre_body(x_ref, id_ref, c1_ref, l1w_ref, l1b_ref, xw_ref, xhat_ref):
    x = x_ref[...]
    nrm = jnp.sqrt(jnp.sum(x * x, axis=1, keepdims=True))
    xn = x / jnp.maximum(nrm, 1e-12)
    xw_ref[...] = _dot(xn, c1_ref[...])
    xhat_ref[...] = _leaky(_dot(xn, l1w_ref[...]) + l1b_ref[...]) + id_ref[...]


def _mid_body(h0_ref, h1_ref, xhat_ref, id_ref, g1w_ref, g1b_ref,
              c2_ref, l2w_ref, l2b_ref, xw2_ref, xhat2_ref):
    h = _leaky(h0_ref[0] + h1_ref[0])
    x2 = _leaky(_dot(h, g1w_ref[...]) + g1b_ref[...] + xhat_ref[...])
    xw2_ref[...] = _dot(x2, c2_ref[...])
    xhat2_ref[...] = _leaky(_dot(x2, l2w_ref[...]) + l2b_ref[...]) + id_ref[...]


def _post_body(h0_ref, h1_ref, xhat2_ref, g2w_ref, g2b_ref, o_ref):
    h = _leaky(h0_ref[0] + h1_ref[0])
    o_ref[...] = _leaky(_dot(h, g2w_ref[...]) + g2b_ref[...] + xhat2_ref[...])


def _row_spec(d):
    return pl.BlockSpec((_BLK, d), lambda i: (i, 0))


def _part_spec(core, d):
    return pl.BlockSpec((1, _BLK, d), lambda i, _c=core: (_c, i, 0))


def _full_spec(r, c):
    return pl.BlockSpec((r, c), lambda i: (0, 0))


_GRID = _N // _BLK

_pre_call = pl.pallas_call(
    _pre_body,
    grid=(_GRID,),
    in_specs=[_row_spec(_DF), _row_spec(_DI), _full_spec(_DF, _DF),
              _full_spec(_DF, _DI), _full_spec(1, _DI)],
    out_specs=[_row_spec(_DF), _row_spec(_DI)],
    out_shape=[jax.ShapeDtypeStruct((_N, _DF), jnp.float32),
               jax.ShapeDtypeStruct((_N, _DI), jnp.float32)],
)

_mid_call = pl.pallas_call(
    _mid_body,
    grid=(_GRID,),
    in_specs=[_part_spec(0, _DF), _part_spec(1, _DF), _row_spec(_DI),
              _row_spec(_DI), _full_spec(_DF, _DI), _full_spec(1, _DI),
              _full_spec(_DI, _DI), _full_spec(_DI, _DI), _full_spec(1, _DI)],
    out_specs=[_row_spec(_DI), _row_spec(_DI)],
    out_shape=[jax.ShapeDtypeStruct((_N, _DI), jnp.float32),
               jax.ShapeDtypeStruct((_N, _DI), jnp.float32)],
)

_post_call = pl.pallas_call(
    _post_body,
    grid=(_GRID,),
    in_specs=[_part_spec(0, _DI), _part_spec(1, _DI), _row_spec(_DI),
              _full_spec(_DI, _DI), _full_spec(1, _DI)],
    out_specs=_row_spec(_DI),
    out_shape=jax.ShapeDtypeStruct((_N, _DI), jnp.float32),
)


def kernel(features, id_embedding, edge_index, conv1_W, lin1_W, lin1_b,
           g1_W, g1_b, conv2_W, lin2_W, lin2_b, g2_W, g2_b):
    ei = edge_index.astype(jnp.int32)
    pad = _E_PAD - _E
    ar = jnp.arange(pad, dtype=jnp.int32)
    src = jnp.concatenate([ei[0], ar % _N]).reshape(_E_PAD // _CHUNK, _CHUNK)
    dst = jnp.concatenate([ei[1], _N + (ar % _PAD_ROWS)]).reshape(
        _E_PAD // _CHUNK, _CHUNK)
    l1b = lin1_b.reshape(1, _DI)
    g1b = g1_b.reshape(1, _DI)
    l2b = lin2_b.reshape(1, _DI)
    g2b = g2_b.reshape(1, _DI)

    xw1, xhat1 = _pre_call(features, id_embedding, conv1_W, lin1_W, l1b)
    hp1 = _sc_scatter_128(xw1, src, dst)
    xw2, xhat2 = _mid_call(hp1, hp1, xhat1, id_embedding,
                           g1_W, g1b, conv2_W, lin2_W, l2b)
    hp2 = _sc_scatter_64(xw2, src, dst)
    out = _post_call(hp2, hp2, xhat2, g2_W, g2b)
    return out


# speedup vs baseline: 1.1373x; 1.0311x over previous
import functools

import jax
import jax.numpy as jnp
from jax import lax
from jax.experimental import pallas as pl
from jax.experimental.pallas import tpu as pltpu
from jax.experimental.pallas import tpu_sc as plsc

_N = 10000
_E = 320000
_DF = 128
_DI = 64

_NC = 2
_NS = 16
_NW = _NC * _NS
_CHUNK = 128
_EPW = 10240
_E_PAD = _NW * _EPW
_PAD_ROWS = 16
_N_ACC = _N + 112
_RPT = _N_ACC // _NS


def _leaky(v):
    return jnp.where(v >= 0, v, 0.01 * v)


def _make_sc_scatter(d, chunk, nbuf):
    nchunk = _EPW // chunk
    hch = 40
    mesh = plsc.VectorSubcoreMesh(core_axis_name="c", subcore_axis_name="s",
                                  num_cores=_NC, num_subcores=_NS)

    params = (pltpu.CompilerParams(use_tc_tiling_on_sc=False)
              if d % 128 != 0 else None)

    @functools.partial(
        pl.kernel,
        out_type=jax.ShapeDtypeStruct((_NC, _N_ACC, d), jnp.float32),
        mesh=mesh,
        compiler_params=params,
        scratch_types=[
            pltpu.VMEM((hch, chunk), jnp.int32),
            pltpu.VMEM((hch, chunk), jnp.int32),
            [pltpu.VMEM((chunk, d), jnp.float32) for _ in range(nbuf)],
            pltpu.VMEM_SHARED((_N_ACC, d), jnp.float32),
            [pltpu.SemaphoreType.DMA for _ in range(nbuf)],
            [pltpu.SemaphoreType.DMA for _ in range(nbuf)],
        ],
    )
    def sc_scatter(xw_hbm, src_hbm, dst_hbm, out_hbm,
                   src_v, dst_v, rows, acc_sh, gsem, ssem):
        c = lax.axis_index("c")
        s = lax.axis_index("s")
        w = c * _NS + s

        def gather(j, b):
            pltpu.async_copy(xw_hbm.at[src_v.at[j]], rows[b], gsem[b])

        def gather_wait(b):
            pltpu.make_async_copy(xw_hbm.at[src_v.at[0]], rows[b],
                                  gsem[b]).wait()

        def scatter(j, b):
            pltpu.async_copy(rows[b], acc_sh.at[dst_v.at[j]], ssem[b],
                             add=True)

        def scatter_wait(b):
            pltpu.make_async_copy(rows[b], acc_sh.at[dst_v.at[0]],
                                  ssem[b]).wait()

        def zbody(r, carry):
            for k in range(d // 16):
                rows[0][r, pl.ds(16 * k, 16)] = jnp.zeros((16,), jnp.float32)
            return carry

        lax.fori_loop(0, chunk, zbody, 0)
        for t in range(_RPT // chunk):
            pltpu.sync_copy(rows[0],
                            acc_sh.at[pl.ds(s * _RPT + t * chunk, chunk)])
        rem = _RPT % chunk
        if rem:
            pltpu.sync_copy(
                rows[0].at[pl.ds(0, rem)],
                acc_sh.at[pl.ds(s * _RPT + (_RPT // chunk) * chunk, rem)])
        plsc.subcore_barrier()

        for p in range(nchunk // hch):
            base = w * nchunk + p * hch
            pltpu.sync_copy(src_hbm.at[pl.ds(base, hch)], src_v)
            pltpu.sync_copy(dst_hbm.at[pl.ds(base, hch)], dst_v)
            if nbuf == 2:
                gather(0, 0)
                gather(1, 1)

                def body(g, carry):
                    j = 2 * g
                    for i in range(2):
                        gather_wait(i)
                        scatter(j + i, i)
                        scatter_wait(i)
                        gather(j + i + 2, i)
                    return carry

                lax.fori_loop(0, hch // 2 - 1, body, 0)
                gather_wait(0)
                scatter(hch - 2, 0)
                gather_wait(1)
                scatter(hch - 1, 1)
                scatter_wait(0)
                scatter_wait(1)
            else:
                gather(0, 0)
                gather(1, 1)
                gather_wait(0)
                scatter(0, 0)
                gather(2, 2)
                gather_wait(1)
                scatter(1, 1)
                gather(3, 3)

                def body(g, carry):
                    j = 4 * g + 2
                    for i in range(4):
                        b = (2 + i) % 4
                        gather_wait(b)
                        scatter(j + i, b)
                        scatter_wait((b + 2) % 4)
                        gather(j + i + 2, (b + 2) % 4)
                    return carry

                lax.fori_loop(0, (hch - 4) // 4, body, 0)
                gather_wait(2)
                scatter(hch - 2, 2)
                gather_wait(3)
                scatter(hch - 1, 3)
                for b in range(4):
                    scatter_wait(b)
        plsc.subcore_barrier()
        pltpu.sync_copy(acc_sh.at[pl.ds(s * _RPT, _RPT)],
                        out_hbm.at[c, pl.ds(s * _RPT, _RPT)])

    return sc_scatter


_sc_scatter_128 = _make_sc_scatter(_DF, _CHUNK, 2)
_sc_scatter_64 = _make_sc_scatter(_DI, _CHUNK, 4)


_BLK = 2000


def _dot(a, b):
    return jnp.dot(a, b, preferred_element_type=jnp.float32)


def _pre_body(x_ref, id_ref, c1_ref, l1w_ref, l1b_ref, xw_ref, xhat_ref):
    x = x_ref[...]
    nrm = jnp.sqrt(jnp.sum(x * x, axis=1, keepdims=True))
    xn = x / jnp.maximum(nrm, 1e-12)
    xw_ref[...] = _dot(xn, c1_ref[...])
    xhat_ref[...] = _leaky(_dot(xn, l1w_ref[...]) + l1b_ref[...]) + id_ref[...]


def _mid_body(h0_ref, h1_ref, xhat_ref, id_ref, g1w_ref, g1b_ref,
              c2_ref, l2w_ref, l2b_ref, xw2_ref, xhat2_ref):
    h = _leaky(h0_ref[0] + h1_ref[0])
    x2 = _leaky(_dot(h, g1w_ref[...]) + g1b_ref[...] + xhat_ref[...])
    xw2_ref[...] = _dot(x2, c2_ref[...])
    xhat2_ref[...] = _leaky(_dot(x2, l2w_ref[...]) + l2b_ref[...]) + id_ref[...]


def _post_body(h0_ref, h1_ref, xhat2_ref, g2w_ref, g2b_ref, o_ref):
    h = _leaky(h0_ref[0] + h1_ref[0])
    o_ref[...] = _leaky(_dot(h, g2w_ref[...]) + g2b_ref[...] + xhat2_ref[...])


def _row_spec(d):
    return pl.BlockSpec((_BLK, d), lambda i: (i, 0))


def _part_spec(core, d):
    return pl.BlockSpec((1, _BLK, d), lambda i, _c=core: (_c, i, 0))


def _full_spec(r, c):
    return pl.BlockSpec((r, c), lambda i: (0, 0))


_GRID = _N // _BLK

_pre_call = pl.pallas_call(
    _pre_body,
    grid=(_GRID,),
    in_specs=[_row_spec(_DF), _row_spec(_DI), _full_spec(_DF, _DF),
              _full_spec(_DF, _DI), _full_spec(1, _DI)],
    out_specs=[_row_spec(_DF), _row_spec(_DI)],
    out_shape=[jax.ShapeDtypeStruct((_N, _DF), jnp.float32),
               jax.ShapeDtypeStruct((_N, _DI), jnp.float32)],
)

_mid_call = pl.pallas_call(
    _mid_body,
    grid=(_GRID,),
    in_specs=[_part_spec(0, _DF), _part_spec(1, _DF), _row_spec(_DI),
              _row_spec(_DI), _full_spec(_DF, _DI), _full_spec(1, _DI),
              _full_spec(_DI, _DI), _full_spec(_DI, _DI), _full_spec(1, _DI)],
    out_specs=[_row_spec(_DI), _row_spec(_DI)],
    out_shape=[jax.ShapeDtypeStruct((_N, _DI), jnp.float32),
               jax.ShapeDtypeStruct((_N, _DI), jnp.float32)],
)

_post_call = pl.pallas_call(
    _post_body,
    grid=(_GRID,),
    in_specs=[_part_spec(0, _DI), _part_spec(1, _DI), _row_spec(_DI),
              _full_spec(_DI, _DI), _full_spec(1, _DI)],
    out_specs=_row_spec(_DI),
    out_shape=jax.ShapeDtypeStruct((_N, _DI), jnp.float32),
)


def kernel(features, id_embedding, edge_index, conv1_W, lin1_W, lin1_b,
           g1_W, g1_b, conv2_W, lin2_W, lin2_b, g2_W, g2_b):
    ei = edge_index.astype(jnp.int32)
    pad = _E_PAD - _E
    ar = jnp.arange(pad, dtype=jnp.int32)
    src = jnp.concatenate([ei[0], ar % _N]).reshape(_E_PAD // _CHUNK, _CHUNK)
    dst = jnp.concatenate([ei[1], _N + (ar % _PAD_ROWS)]).reshape(
        _E_PAD // _CHUNK, _CHUNK)
    l1b = lin1_b.reshape(1, _DI)
    g1b = g1_b.reshape(1, _DI)
    l2b = lin2_b.reshape(1, _DI)
    g2b = g2_b.reshape(1, _DI)

    xw1, xhat1 = _pre_call(features, id_embedding, conv1_W, lin1_W, l1b)
    hp1 = _sc_scatter_128(xw1, src, dst)
    xw2, xhat2 = _mid_call(hp1, hp1, xhat1, id_embedding,
                           g1_W, g1b, conv2_W, lin2_W, l2b)
    hp2 = _sc_scatter_64(xw2, src, dst)
    out = _post_call(hp2, hp2, xhat2, g2_W, g2b)
    return out


# speedup vs baseline: 1.1907x; 1.0470x over previous
import functools

import jax
import jax.numpy as jnp
from jax import lax
from jax.experimental import pallas as pl
from jax.experimental.pallas import tpu as pltpu
from jax.experimental.pallas import tpu_sc as plsc

_N = 10000
_E = 320000
_DF = 128
_DI = 64

_NC = 2
_NS = 16
_NW = _NC * _NS
_CHUNK = 128
_EPW = 10240
_E_PAD = _NW * _EPW
_PAD_ROWS = 16
_N_ACC = _N + 112
_RPT = _N_ACC // _NS


def _leaky(v):
    return jnp.where(v >= 0, v, 0.01 * v)


def _make_sc_scatter(d, chunk, nbuf):
    nchunk = _EPW // chunk
    hch = 40
    mesh = plsc.VectorSubcoreMesh(core_axis_name="c", subcore_axis_name="s",
                                  num_cores=_NC, num_subcores=_NS)

    params = (pltpu.CompilerParams(use_tc_tiling_on_sc=False)
              if d % 128 != 0 else None)

    @functools.partial(
        pl.kernel,
        out_type=jax.ShapeDtypeStruct((_NC, _N_ACC, d), jnp.float32),
        mesh=mesh,
        compiler_params=params,
        scratch_types=[
            pltpu.VMEM((hch, chunk), jnp.int32),
            pltpu.VMEM((hch, chunk), jnp.int32),
            [pltpu.VMEM((chunk, d), jnp.float32) for _ in range(nbuf)],
            pltpu.VMEM_SHARED((_N_ACC, d), jnp.float32),
            [pltpu.SemaphoreType.DMA for _ in range(nbuf)],
            [pltpu.SemaphoreType.DMA for _ in range(nbuf)],
        ],
    )
    def sc_scatter(xw_hbm, src_hbm, dst_hbm, out_hbm,
                   src_v, dst_v, rows, acc_sh, gsem, ssem):
        c = lax.axis_index("c")
        s = lax.axis_index("s")
        w = c * _NS + s

        def gather(j, b):
            pltpu.async_copy(xw_hbm.at[src_v.at[j]], rows[b], gsem[b])

        def gather_wait(b):
            pltpu.make_async_copy(xw_hbm.at[src_v.at[0]], rows[b],
                                  gsem[b]).wait()

        def scatter(j, b):
            pltpu.async_copy(rows[b], acc_sh.at[dst_v.at[j]], ssem[b],
                             add=True)

        def scatter_wait(b):
            pltpu.make_async_copy(rows[b], acc_sh.at[dst_v.at[0]],
                                  ssem[b]).wait()

        def zbody(r, carry):
            for k in range(d // 16):
                rows[0][r, pl.ds(16 * k, 16)] = jnp.zeros((16,), jnp.float32)
            return carry

        lax.fori_loop(0, chunk, zbody, 0)
        for t in range(_RPT // chunk):
            pltpu.sync_copy(rows[0],
                            acc_sh.at[pl.ds(s * _RPT + t * chunk, chunk)])
        rem = _RPT % chunk
        if rem:
            pltpu.sync_copy(
                rows[0].at[pl.ds(0, rem)],
                acc_sh.at[pl.ds(s * _RPT + (_RPT // chunk) * chunk, rem)])
        plsc.subcore_barrier()

        for p in range(nchunk // hch):
            base = w * nchunk + p * hch
            pltpu.sync_copy(src_hbm.at[pl.ds(base, hch)], src_v)
            pltpu.sync_copy(dst_hbm.at[pl.ds(base, hch)], dst_v)
            if nbuf == 2:
                gather(0, 0)
                gather(1, 1)

                def body(g, carry):
                    j = 2 * g
                    for i in range(2):
                        gather_wait(i)
                        scatter(j + i, i)
                        scatter_wait(i)
                        gather(j + i + 2, i)
                    return carry

                lax.fori_loop(0, hch // 2 - 1, body, 0)
                gather_wait(0)
                scatter(hch - 2, 0)
                gather_wait(1)
                scatter(hch - 1, 1)
                scatter_wait(0)
                scatter_wait(1)
            else:
                gather(0, 0)
                gather(1, 1)
                gather_wait(0)
                scatter(0, 0)
                gather(2, 2)
                gather_wait(1)
                scatter(1, 1)
                gather(3, 3)

                def body(g, carry):
                    j = 4 * g + 2
                    for i in range(4):
                        b = (2 + i) % 4
                        gather_wait(b)
                        scatter(j + i, b)
                        scatter_wait((b + 2) % 4)
                        gather(j + i + 2, (b + 2) % 4)
                    return carry

                lax.fori_loop(0, (hch - 4) // 4, body, 0)
                gather_wait(2)
                scatter(hch - 2, 2)
                gather_wait(3)
                scatter(hch - 1, 3)
                for b in range(4):
                    scatter_wait(b)
        plsc.subcore_barrier()
        pltpu.sync_copy(acc_sh.at[pl.ds(s * _RPT, _RPT)],
                        out_hbm.at[c, pl.ds(s * _RPT, _RPT)])

    return sc_scatter


_sc_scatter_128 = _make_sc_scatter(_DF, _CHUNK, 2)
_sc_scatter_64 = _make_sc_scatter(_DI, _CHUNK, 4)


_BLK = 2000


def _dot(a, b):
    return jnp.dot(a, b, preferred_element_type=jnp.float32)


_EBLK = _E_PAD // (_N // _BLK)
_ROWS_PER_STEP = _EBLK // _CHUNK


def _pre_body(x_ref, id_ref, c1_ref, l1w_ref, l1b_ref, ei_ref,
              xw_ref, xhat_ref, src_ref, dst_ref):
    x = x_ref[...]
    nrm = jnp.sqrt(jnp.sum(x * x, axis=1, keepdims=True))
    xn = x / jnp.maximum(nrm, 1e-12)
    xw_ref[...] = _dot(xn, c1_ref[...])
    xhat_ref[...] = _leaky(_dot(xn, l1w_ref[...]) + l1b_ref[...]) + id_ref[...]
    i = pl.program_id(0)
    eid = (i * _EBLK
           + lax.broadcasted_iota(jnp.int32, (_ROWS_PER_STEP, _CHUNK), 0)
           * _CHUNK
           + lax.broadcasted_iota(jnp.int32, (_ROWS_PER_STEP, _CHUNK), 1))
    real = eid < _E
    s_in = ei_ref[0].reshape(_ROWS_PER_STEP, _CHUNK)
    d_in = ei_ref[1].reshape(_ROWS_PER_STEP, _CHUNK)
    src_ref[...] = jnp.where(real, s_in, eid - _E)
    dst_ref[...] = jnp.where(real, d_in, _N + ((eid - _E) & (_PAD_ROWS - 1)))


def _mid_body(h0_ref, h1_ref, xhat_ref, id_ref, g1w_ref, g1b_ref,
              c2_ref, l2w_ref, l2b_ref, xw2_ref, xhat2_ref):
    h = _leaky(h0_ref[0] + h1_ref[0])
    x2 = _leaky(_dot(h, g1w_ref[...]) + g1b_ref[...] + xhat_ref[...])
    xw2_ref[...] = _dot(x2, c2_ref[...])
    xhat2_ref[...] = _leaky(_dot(x2, l2w_ref[...]) + l2b_ref[...]) + id_ref[...]


def _post_body(h0_ref, h1_ref, xhat2_ref, g2w_ref, g2b_ref, o_ref):
    h = _leaky(h0_ref[0] + h1_ref[0])
    o_ref[...] = _leaky(_dot(h, g2w_ref[...]) + g2b_ref[...] + xhat2_ref[...])


def _row_spec(d):
    return pl.BlockSpec((_BLK, d), lambda i: (i, 0))


def _part_spec(core, d):
    return pl.BlockSpec((1, _BLK, d), lambda i, _c=core: (_c, i, 0))


def _full_spec(r, c):
    return pl.BlockSpec((r, c), lambda i: (0, 0))


_GRID = _N // _BLK

_pre_call = pl.pallas_call(
    _pre_body,
    grid=(_GRID,),
    in_specs=[_row_spec(_DF), _row_spec(_DI), _full_spec(_DF, _DF),
              _full_spec(_DF, _DI), _full_spec(1, _DI),
              pl.BlockSpec((2, _EBLK), lambda i: (0, i))],
    out_specs=[_row_spec(_DF), _row_spec(_DI),
               pl.BlockSpec((_ROWS_PER_STEP, _CHUNK), lambda i: (i, 0)),
               pl.BlockSpec((_ROWS_PER_STEP, _CHUNK), lambda i: (i, 0))],
    out_shape=[jax.ShapeDtypeStruct((_N, _DF), jnp.float32),
               jax.ShapeDtypeStruct((_N, _DI), jnp.float32),
               jax.ShapeDtypeStruct((_E_PAD // _CHUNK, _CHUNK), jnp.int32),
               jax.ShapeDtypeStruct((_E_PAD // _CHUNK, _CHUNK), jnp.int32)],
)

_mid_call = pl.pallas_call(
    _mid_body,
    grid=(_GRID,),
    in_specs=[_part_spec(0, _DF), _part_spec(1, _DF), _row_spec(_DI),
              _row_spec(_DI), _full_spec(_DF, _DI), _full_spec(1, _DI),
              _full_spec(_DI, _DI), _full_spec(_DI, _DI), _full_spec(1, _DI)],
    out_specs=[_row_spec(_DI), _row_spec(_DI)],
    out_shape=[jax.ShapeDtypeStruct((_N, _DI), jnp.float32),
               jax.ShapeDtypeStruct((_N, _DI), jnp.float32)],
)

_post_call = pl.pallas_call(
    _post_body,
    grid=(_GRID,),
    in_specs=[_part_spec(0, _DI), _part_spec(1, _DI), _row_spec(_DI),
              _full_spec(_DI, _DI), _full_spec(1, _DI)],
    out_specs=_row_spec(_DI),
    out_shape=jax.ShapeDtypeStruct((_N, _DI), jnp.float32),
)


def kernel(features, id_embedding, edge_index, conv1_W, lin1_W, lin1_b,
           g1_W, g1_b, conv2_W, lin2_W, lin2_b, g2_W, g2_b):
    ei = edge_index.astype(jnp.int32)
    l1b = lin1_b.reshape(1, _DI)
    g1b = g1_b.reshape(1, _DI)
    l2b = lin2_b.reshape(1, _DI)
    g2b = g2_b.reshape(1, _DI)

    xw1, xhat1, src, dst = _pre_call(features, id_embedding, conv1_W,
                                     lin1_W, l1b, ei)
    hp1 = _sc_scatter_128(xw1, src, dst)
    xw2, xhat2 = _mid_call(hp1, hp1, xhat1, id_embedding,
                           g1_W, g1b, conv2_W, lin2_W, l2b)
    hp2 = _sc_scatter_64(xw2, src, dst)
    out = _post_call(hp2, hp2, xhat2, g2_W, g2b)
    return out


# speedup vs baseline: 1.2299x; 1.0329x over previous
import functools

import jax
import jax.numpy as jnp
from jax import lax
from jax.experimental import pallas as pl
from jax.experimental.pallas import tpu as pltpu
from jax.experimental.pallas import tpu_sc as plsc

_N = 10000
_E = 320000
_DF = 128
_DI = 64

_NC = 2
_NS = 16
_NW = _NC * _NS
_CHUNK = 128
_EPW = 10240
_E_PAD = _NW * _EPW
_PAD_ROWS = 16
_N_ACC = _N + 112
_RPT = _N_ACC // _NS


def _leaky(v):
    return jnp.where(v >= 0, v, 0.01 * v)


def _make_sc_scatter(d, chunk, nbuf):
    nchunk = _EPW // chunk
    hch = 40
    mesh = plsc.VectorSubcoreMesh(core_axis_name="c", subcore_axis_name="s",
                                  num_cores=_NC, num_subcores=_NS)

    params = (pltpu.CompilerParams(use_tc_tiling_on_sc=False)
              if d % 128 != 0 else None)

    @functools.partial(
        pl.kernel,
        out_type=jax.ShapeDtypeStruct((_NC, _N_ACC, d), jnp.float32),
        mesh=mesh,
        compiler_params=params,
        scratch_types=[
            pltpu.VMEM((hch, chunk), jnp.int32),
            pltpu.VMEM((hch, chunk), jnp.int32),
            [pltpu.VMEM((chunk, d), jnp.float32) for _ in range(nbuf)],
            pltpu.VMEM_SHARED((_N_ACC, d), jnp.float32),
            [pltpu.SemaphoreType.DMA for _ in range(nbuf)],
            [pltpu.SemaphoreType.DMA for _ in range(nbuf)],
        ],
    )
    def sc_scatter(xw_hbm, src_hbm, dst_hbm, out_hbm,
                   src_v, dst_v, rows, acc_sh, gsem, ssem):
        c = lax.axis_index("c")
        s = lax.axis_index("s")
        w = c * _NS + s

        def gather(j, b):
            pltpu.async_copy(xw_hbm.at[src_v.at[j]], rows[b], gsem[b])

        def gather_wait(b):
            pltpu.make_async_copy(xw_hbm.at[src_v.at[0]], rows[b],
                                  gsem[b]).wait()

        def scatter(j, b):
            pltpu.async_copy(rows[b], acc_sh.at[dst_v.at[j]], ssem[b],
                             add=True)

        def scatter_wait(b):
            pltpu.make_async_copy(rows[b], acc_sh.at[dst_v.at[0]],
                                  ssem[b]).wait()

        def zbody(r, carry):
            for k in range(d // 16):
                rows[0][r, pl.ds(16 * k, 16)] = jnp.zeros((16,), jnp.float32)
            return carry

        lax.fori_loop(0, chunk, zbody, 0)
        for t in range(_RPT // chunk):
            pltpu.sync_copy(rows[0],
                            acc_sh.at[pl.ds(s * _RPT + t * chunk, chunk)])
        rem = _RPT % chunk
        if rem:
            pltpu.sync_copy(
                rows[0].at[pl.ds(0, rem)],
                acc_sh.at[pl.ds(s * _RPT + (_RPT // chunk) * chunk, rem)])
        plsc.subcore_barrier()

        for p in range(nchunk // hch):
            base = w * nchunk + p * hch
            pltpu.sync_copy(src_hbm.at[pl.ds(base, hch)], src_v)
            pltpu.sync_copy(dst_hbm.at[pl.ds(base, hch)], dst_v)
            if nbuf == 2:
                gather(0, 0)
                gather(1, 1)

                def body(g, carry):
                    j = 2 * g
                    for i in range(2):
                        gather_wait(i)
                        scatter(j + i, i)
                        scatter_wait(i)
                        gather(j + i + 2, i)
                    return carry

                lax.fori_loop(0, hch // 2 - 1, body, 0)
                gather_wait(0)
                scatter(hch - 2, 0)
                gather_wait(1)
                scatter(hch - 1, 1)
                scatter_wait(0)
                scatter_wait(1)
            else:
                gather(0, 0)
                gather(1, 1)
                gather_wait(0)
                scatter(0, 0)
                gather(2, 2)
                gather_wait(1)
                scatter(1, 1)
                gather(3, 3)

                def body(g, carry):
                    j = 4 * g + 2
                    for i in range(4):
                        b = (2 + i) % 4
                        gather_wait(b)
                        scatter(j + i, b)
                        scatter_wait((b + 2) % 4)
                        gather(j + i + 2, (b + 2) % 4)
                    return carry

                lax.fori_loop(0, (hch - 4) // 4, body, 0)
                gather_wait(2)
                scatter(hch - 2, 2)
                gather_wait(3)
                scatter(hch - 1, 3)
                for b in range(4):
                    scatter_wait(b)
        plsc.subcore_barrier()
        pltpu.sync_copy(acc_sh.at[pl.ds(s * _RPT, _RPT)],
                        out_hbm.at[c, pl.ds(s * _RPT, _RPT)])

    return sc_scatter


_sc_scatter_128 = _make_sc_scatter(_DF, _CHUNK, 2)
_sc_scatter_64 = _make_sc_scatter(_DI, _CHUNK, 4)


_BLK = 2000


def _dot(a, b):
    return jnp.dot(a, b, preferred_element_type=jnp.float32)


_EBLK = _E_PAD // (_N // _BLK)
_ROWS_PER_STEP = _EBLK // _CHUNK


def _normed(x_ref):
    x = x_ref[...]
    nrm = jnp.sqrt(jnp.sum(x * x, axis=1, keepdims=True))
    return x / jnp.maximum(nrm, 1e-12)


def _pre_a_body(x_ref, c1_ref, ei_ref, xw_ref, src_ref, dst_ref):
    xw_ref[...] = _dot(_normed(x_ref), c1_ref[...])
    i = pl.program_id(0)
    eid = (i * _EBLK
           + lax.broadcasted_iota(jnp.int32, (_ROWS_PER_STEP, _CHUNK), 0)
           * _CHUNK
           + lax.broadcasted_iota(jnp.int32, (_ROWS_PER_STEP, _CHUNK), 1))
    real = eid < _E
    s_in = ei_ref[0].reshape(_ROWS_PER_STEP, _CHUNK)
    d_in = ei_ref[1].reshape(_ROWS_PER_STEP, _CHUNK)
    src_ref[...] = jnp.where(real, s_in, eid - _E)
    dst_ref[...] = jnp.where(real, d_in, _N + ((eid - _E) & (_PAD_ROWS - 1)))


def _pre_b_body(x_ref, id_ref, l1w_ref, l1b_ref, xhat_ref):
    xn = _normed(x_ref)
    xhat_ref[...] = _leaky(_dot(xn, l1w_ref[...]) + l1b_ref[...]) + id_ref[...]


def _mid_a_body(h0_ref, h1_ref, xhat_ref, g1w_ref, g1b_ref, c2_ref,
                xw2_ref, x2_ref):
    h = _leaky(h0_ref[0] + h1_ref[0])
    x2 = _leaky(_dot(h, g1w_ref[...]) + g1b_ref[...] + xhat_ref[...])
    xw2_ref[...] = _dot(x2, c2_ref[...])
    x2_ref[...] = x2


def _mid_b_body(x2_ref, id_ref, l2w_ref, l2b_ref, xhat2_ref):
    xhat2_ref[...] = (_leaky(_dot(x2_ref[...], l2w_ref[...]) + l2b_ref[...])
                      + id_ref[...])


def _post_body(h0_ref, h1_ref, xhat2_ref, g2w_ref, g2b_ref, o_ref):
    h = _leaky(h0_ref[0] + h1_ref[0])
    o_ref[...] = _leaky(_dot(h, g2w_ref[...]) + g2b_ref[...] + xhat2_ref[...])


def _row_spec(d):
    return pl.BlockSpec((_BLK, d), lambda i: (i, 0))


def _part_spec(core, d):
    return pl.BlockSpec((1, _BLK, d), lambda i, _c=core: (_c, i, 0))


def _full_spec(r, c):
    return pl.BlockSpec((r, c), lambda i: (0, 0))


_GRID = _N // _BLK

_pre_a_call = pl.pallas_call(
    _pre_a_body,
    grid=(_GRID,),
    in_specs=[_row_spec(_DF), _full_spec(_DF, _DF),
              pl.BlockSpec((2, _EBLK), lambda i: (0, i))],
    out_specs=[_row_spec(_DF),
               pl.BlockSpec((_ROWS_PER_STEP, _CHUNK), lambda i: (i, 0)),
               pl.BlockSpec((_ROWS_PER_STEP, _CHUNK), lambda i: (i, 0))],
    out_shape=[jax.ShapeDtypeStruct((_N, _DF), jnp.float32),
               jax.ShapeDtypeStruct((_E_PAD // _CHUNK, _CHUNK), jnp.int32),
               jax.ShapeDtypeStruct((_E_PAD // _CHUNK, _CHUNK), jnp.int32)],
)

_pre_b_call = pl.pallas_call(
    _pre_b_body,
    grid=(_GRID,),
    in_specs=[_row_spec(_DF), _row_spec(_DI), _full_spec(_DF, _DI),
              _full_spec(1, _DI)],
    out_specs=_row_spec(_DI),
    out_shape=jax.ShapeDtypeStruct((_N, _DI), jnp.float32),
)

_mid_a_call = pl.pallas_call(
    _mid_a_body,
    grid=(_GRID,),
    in_specs=[_part_spec(0, _DF), _part_spec(1, _DF), _row_spec(_DI),
              _full_spec(_DF, _DI), _full_spec(1, _DI),
              _full_spec(_DI, _DI)],
    out_specs=[_row_spec(_DI), _row_spec(_DI)],
    out_shape=[jax.ShapeDtypeStruct((_N, _DI), jnp.float32),
               jax.ShapeDtypeStruct((_N, _DI), jnp.float32)],
)

_mid_b_call = pl.pallas_call(
    _mid_b_body,
    grid=(_GRID,),
    in_specs=[_row_spec(_DI), _row_spec(_DI), _full_spec(_DI, _DI),
              _full_spec(1, _DI)],
    out_specs=_row_spec(_DI),
    out_shape=jax.ShapeDtypeStruct((_N, _DI), jnp.float32),
)

_post_call = pl.pallas_call(
    _post_body,
    grid=(_GRID,),
    in_specs=[_part_spec(0, _DI), _part_spec(1, _DI), _row_spec(_DI),
              _full_spec(_DI, _DI), _full_spec(1, _DI)],
    out_specs=_row_spec(_DI),
    out_shape=jax.ShapeDtypeStruct((_N, _DI), jnp.float32),
)


def kernel(features, id_embedding, edge_index, conv1_W, lin1_W, lin1_b,
           g1_W, g1_b, conv2_W, lin2_W, lin2_b, g2_W, g2_b):
    ei = edge_index.astype(jnp.int32)
    l1b = lin1_b.reshape(1, _DI)
    g1b = g1_b.reshape(1, _DI)
    l2b = lin2_b.reshape(1, _DI)
    g2b = g2_b.reshape(1, _DI)

    xw1, src, dst = _pre_a_call(features, conv1_W, ei)
    hp1 = _sc_scatter_128(xw1, src, dst)
    xhat1 = _pre_b_call(features, id_embedding, lin1_W, l1b)
    xw2, x2 = _mid_a_call(hp1, hp1, xhat1, g1_W, g1b, conv2_W)
    hp2 = _sc_scatter_64(xw2, src, dst)
    xhat2 = _mid_b_call(x2, id_embedding, lin2_W, l2b)
    out = _post_call(hp2, hp2, xhat2, g2_W, g2b)
    return out


# speedup vs baseline: 1.2388x; 1.0073x over previous
import functools

import jax
import jax.numpy as jnp
from jax import lax
from jax.experimental import pallas as pl
from jax.experimental.pallas import tpu as pltpu
from jax.experimental.pallas import tpu_sc as plsc

_N = 10000
_E = 320000
_DF = 128
_DI = 64

_NC = 2
_NS = 16
_NW = _NC * _NS
_CHUNK = 128
_EPW = 10240
_E_PAD = _NW * _EPW
_PAD_ROWS = 16
_N_ACC = _N + 112
_RPT = _N_ACC // _NS


def _leaky(v):
    return jnp.where(v >= 0, v, 0.01 * v)


def _make_sc_scatter(d, chunk, nbuf, hch):
    nchunk = _EPW // chunk
    mesh = plsc.VectorSubcoreMesh(core_axis_name="c", subcore_axis_name="s",
                                  num_cores=_NC, num_subcores=_NS)

    params = (pltpu.CompilerParams(use_tc_tiling_on_sc=False)
              if d % 128 != 0 else None)

    @functools.partial(
        pl.kernel,
        out_type=jax.ShapeDtypeStruct((_NC, _N_ACC, d), jnp.float32),
        mesh=mesh,
        compiler_params=params,
        scratch_types=[
            pltpu.VMEM((hch, chunk), jnp.int32),
            pltpu.VMEM((hch, chunk), jnp.int32),
            [pltpu.VMEM((chunk, d), jnp.float32) for _ in range(nbuf)],
            pltpu.VMEM_SHARED((_N_ACC, d), jnp.float32),
            [pltpu.SemaphoreType.DMA for _ in range(nbuf)],
            [pltpu.SemaphoreType.DMA for _ in range(nbuf)],
        ],
    )
    def sc_scatter(xw_hbm, src_hbm, dst_hbm, out_hbm,
                   src_v, dst_v, rows, acc_sh, gsem, ssem):
        c = lax.axis_index("c")
        s = lax.axis_index("s")
        w = c * _NS + s

        def gather(j, b):
            pltpu.async_copy(xw_hbm.at[src_v.at[j]], rows[b], gsem[b])

        def gather_wait(b):
            pltpu.make_async_copy(xw_hbm.at[src_v.at[0]], rows[b],
                                  gsem[b]).wait()

        def scatter(j, b):
            pltpu.async_copy(rows[b], acc_sh.at[dst_v.at[j]], ssem[b],
                             add=True)

        def scatter_wait(b):
            pltpu.make_async_copy(rows[b], acc_sh.at[dst_v.at[0]],
                                  ssem[b]).wait()

        def zbody(r, carry):
            for k in range(d // 16):
                rows[0][r, pl.ds(16 * k, 16)] = jnp.zeros((16,), jnp.float32)
            return carry

        lax.fori_loop(0, chunk, zbody, 0)
        for t in range(_RPT // chunk):
            pltpu.sync_copy(rows[0],
                            acc_sh.at[pl.ds(s * _RPT + t * chunk, chunk)])
        rem = _RPT % chunk
        if rem:
            pltpu.sync_copy(
                rows[0].at[pl.ds(0, rem)],
                acc_sh.at[pl.ds(s * _RPT + (_RPT // chunk) * chunk, rem)])
        plsc.subcore_barrier()

        for p in range(nchunk // hch):
            base = w * nchunk + p * hch
            pltpu.sync_copy(src_hbm.at[pl.ds(base, hch)], src_v)
            pltpu.sync_copy(dst_hbm.at[pl.ds(base, hch)], dst_v)
            if nbuf == 2:
                gather(0, 0)
                gather(1, 1)

                def body(g, carry):
                    j = 2 * g
                    for i in range(2):
                        gather_wait(i)
                        scatter(j + i, i)
                        scatter_wait(i)
                        gather(j + i + 2, i)
                    return carry

                lax.fori_loop(0, hch // 2 - 1, body, 0)
                gather_wait(0)
                scatter(hch - 2, 0)
                gather_wait(1)
                scatter(hch - 1, 1)
                scatter_wait(0)
                scatter_wait(1)
            else:
                gather(0, 0)
                gather(1, 1)
                gather_wait(0)
                scatter(0, 0)
                gather(2, 2)
                gather_wait(1)
                scatter(1, 1)
                gather(3, 3)

                def body(g, carry):
                    j = 4 * g + 2
                    for i in range(4):
                        b = (2 + i) % 4
                        gather_wait(b)
                        scatter(j + i, b)
                        scatter_wait((b + 2) % 4)
                        gather(j + i + 2, (b + 2) % 4)
                    return carry

                lax.fori_loop(0, (hch - 4) // 4, body, 0)
                gather_wait(2)
                scatter(hch - 2, 2)
                gather_wait(3)
                scatter(hch - 1, 3)
                for b in range(4):
                    scatter_wait(b)
        plsc.subcore_barrier()
        pltpu.sync_copy(acc_sh.at[pl.ds(s * _RPT, _RPT)],
                        out_hbm.at[c, pl.ds(s * _RPT, _RPT)])

    return sc_scatter


_sc_scatter_128 = _make_sc_scatter(_DF, _CHUNK, 2, 40)
_sc_scatter_64 = _make_sc_scatter(_DI, _CHUNK, 4, 80)


_BLK = 2000


def _dot(a, b):
    return jnp.dot(a, b, preferred_element_type=jnp.float32)


_EBLK = _E_PAD // (_N // _BLK)
_ROWS_PER_STEP = _EBLK // _CHUNK


def _normed(x_ref):
    x = x_ref[...]
    nrm = jnp.sqrt(jnp.sum(x * x, axis=1, keepdims=True))
    return x / jnp.maximum(nrm, 1e-12)


def _pre_a_body(x_ref, c1_ref, ei_ref, xw_ref, src_ref, dst_ref):
    xw_ref[...] = _dot(_normed(x_ref), c1_ref[...])
    i = pl.program_id(0)
    eid = (i * _EBLK
           + lax.broadcasted_iota(jnp.int32, (_ROWS_PER_STEP, _CHUNK), 0)
           * _CHUNK
           + lax.broadcasted_iota(jnp.int32, (_ROWS_PER_STEP, _CHUNK), 1))
    real = eid < _E
    s_in = ei_ref[0].reshape(_ROWS_PER_STEP, _CHUNK)
    d_in = ei_ref[1].reshape(_ROWS_PER_STEP, _CHUNK)
    src_ref[...] = jnp.where(real, s_in, eid - _E)
    dst_ref[...] = jnp.where(real, d_in, _N + ((eid - _E) & (_PAD_ROWS - 1)))


def _pre_b_body(x_ref, id_ref, l1w_ref, l1b_ref, xhat_ref):
    xn = _normed(x_ref)
    xhat_ref[...] = _leaky(_dot(xn, l1w_ref[...]) + l1b_ref[...]) + id_ref[...]


def _mid_a_body(h0_ref, h1_ref, xhat_ref, g1w_ref, g1b_ref, c2_ref,
                xw2_ref, x2_ref):
    h = _leaky(h0_ref[0] + h1_ref[0])
    x2 = _leaky(_dot(h, g1w_ref[...]) + g1b_ref[...] + xhat_ref[...])
    xw2_ref[...] = _dot(x2, c2_ref[...])
    x2_ref[...] = x2


def _mid_b_body(x2_ref, id_ref, l2w_ref, l2b_ref, xhat2_ref):
    xhat2_ref[...] = (_leaky(_dot(x2_ref[...], l2w_ref[...]) + l2b_ref[...])
                      + id_ref[...])


def _post_body(h0_ref, h1_ref, xhat2_ref, g2w_ref, g2b_ref, o_ref):
    h = _leaky(h0_ref[0] + h1_ref[0])
    o_ref[...] = _leaky(_dot(h, g2w_ref[...]) + g2b_ref[...] + xhat2_ref[...])


def _row_spec(d):
    return pl.BlockSpec((_BLK, d), lambda i: (i, 0))


def _part_spec(core, d):
    return pl.BlockSpec((1, _BLK, d), lambda i, _c=core: (_c, i, 0))


def _full_spec(r, c):
    return pl.BlockSpec((r, c), lambda i: (0, 0))


_GRID = _N // _BLK

_pre_a_call = pl.pallas_call(
    _pre_a_body,
    grid=(_GRID,),
    in_specs=[_row_spec(_DF), _full_spec(_DF, _DF),
              pl.BlockSpec((2, _EBLK), lambda i: (0, i))],
    out_specs=[_row_spec(_DF),
               pl.BlockSpec((_ROWS_PER_STEP, _CHUNK), lambda i: (i, 0)),
               pl.BlockSpec((_ROWS_PER_STEP, _CHUNK), lambda i: (i, 0))],
    out_shape=[jax.ShapeDtypeStruct((_N, _DF), jnp.float32),
               jax.ShapeDtypeStruct((_E_PAD // _CHUNK, _CHUNK), jnp.int32),
               jax.ShapeDtypeStruct((_E_PAD // _CHUNK, _CHUNK), jnp.int32)],
)

_pre_b_call = pl.pallas_call(
    _pre_b_body,
    grid=(_GRID,),
    in_specs=[_row_spec(_DF), _row_spec(_DI), _full_spec(_DF, _DI),
              _full_spec(1, _DI)],
    out_specs=_row_spec(_DI),
    out_shape=jax.ShapeDtypeStruct((_N, _DI), jnp.float32),
)

_mid_a_call = pl.pallas_call(
    _mid_a_body,
    grid=(_GRID,),
    in_specs=[_part_spec(0, _DF), _part_spec(1, _DF), _row_spec(_DI),
              _full_spec(_DF, _DI), _full_spec(1, _DI),
              _full_spec(_DI, _DI)],
    out_specs=[_row_spec(_DI), _row_spec(_DI)],
    out_shape=[jax.ShapeDtypeStruct((_N, _DI), jnp.float32),
               jax.ShapeDtypeStruct((_N, _DI), jnp.float32)],
)

_mid_b_call = pl.pallas_call(
    _mid_b_body,
    grid=(_GRID,),
    in_specs=[_row_spec(_DI), _row_spec(_DI), _full_spec(_DI, _DI),
              _full_spec(1, _DI)],
    out_specs=_row_spec(_DI),
    out_shape=jax.ShapeDtypeStruct((_N, _DI), jnp.float32),
)

_post_call = pl.pallas_call(
    _post_body,
    grid=(_GRID,),
    in_specs=[_part_spec(0, _DI), _part_spec(1, _DI), _row_spec(_DI),
              _full_spec(_DI, _DI), _full_spec(1, _DI)],
    out_specs=_row_spec(_DI),
    out_shape=jax.ShapeDtypeStruct((_N, _DI), jnp.float32),
)


def kernel(features, id_embedding, edge_index, conv1_W, lin1_W, lin1_b,
           g1_W, g1_b, conv2_W, lin2_W, lin2_b, g2_W, g2_b):
    ei = edge_index.astype(jnp.int32)
    l1b = lin1_b.reshape(1, _DI)
    g1b = g1_b.reshape(1, _DI)
    l2b = lin2_b.reshape(1, _DI)
    g2b = g2_b.reshape(1, _DI)

    xw1, src, dst = _pre_a_call(features, conv1_W, ei)
    hp1 = _sc_scatter_128(xw1, src, dst)
    xhat1 = _pre_b_call(features, id_embedding, lin1_W, l1b)
    xw2, x2 = _mid_a_call(hp1, hp1, xhat1, g1_W, g1b, conv2_W)
    hp2 = _sc_scatter_64(xw2, src, dst)
    xhat2 = _mid_b_call(x2, id_embedding, lin2_W, l2b)
    out = _post_call(hp2, hp2, xhat2, g2_W, g2b)
    return out
